# Initial kernel scaffold; baseline (speedup 1.0000x reference)
import jax
import jax.numpy as jnp
from jax import lax
from jax.experimental import pallas as pl
from jax.experimental.pallas import tpu as pltpu
from jax.experimental.pallas import tpu_sc as plsc

E = 6_400_000
GROW = 100
CHUNK = 2000
GPC = CHUNK // GROW
NW = 32
EPW = E // NW
CPW = EPW // CHUNK
VI = CHUNK // 16


def _body(tab_hbm, src_hbm, dst_hbm, sig1_hbm, sig2_hbm, out_hbm,
          idx1_v, idx2_v, rows1_v, rows2_v, out_v, sig1_v, sig2_v, sem):
    nc = 2
    wid = lax.axis_index("s") * nc + lax.axis_index("c")

    pltpu.sync_copy(sig1_hbm, sig1_v)
    pltpu.sync_copy(sig2_hbm, sig2_v)
    s1 = sig1_v[...]
    s2 = sig2_v[...]
    c1 = s1 * s1
    c2 = s2 * s2

    def chunk_body(c, carry):
        row0 = wid * (CPW * GPC) + c * GPC
        pltpu.sync_copy(src_hbm.at[pl.ds(row0, GPC)], idx1_v)
        pltpu.sync_copy(dst_hbm.at[pl.ds(row0, GPC)], idx2_v)
        copies = []
        for g in range(GPC):
            copies.append(pltpu.async_copy(
                tab_hbm.at[idx1_v.at[g]], rows1_v.at[pl.ds(g * GROW, GROW)], sem))
            copies.append(pltpu.async_copy(
                tab_hbm.at[idx2_v.at[g]], rows2_v.at[pl.ds(g * GROW, GROW)], sem))
        for d in copies:
            d.wait()

        def vec_body(i, carry2):
            rows = i * 16 + lax.iota(jnp.int32, 16)

            def col(rv, j):
                return plsc.load_gather(rv, [rows, jnp.full((16,), j, jnp.int32)])

            dx = col(rows1_v, 0) - col(rows2_v, 0)
            dy = col(rows1_v, 1) - col(rows2_v, 1)
            dz = col(rows1_v, 2) - col(rows2_v, 2)
            D = dx * dx + dy * dy + dz * dz
            dot = (col(rows1_v, 3) * col(rows2_v, 3)
                   + col(rows1_v, 4) * col(rows2_v, 4)
                   + col(rows1_v, 5) * col(rows2_v, 5))
            t = 1.0 - dot
            T = t * t
            out_v[pl.ds(i * 16, 16)] = jnp.exp(-(D * c1 + T * c2))
            return carry2

        lax.fori_loop(0, VI, vec_body, 0)
        pltpu.sync_copy(out_v, out_hbm.at[pl.ds(row0 * GROW, CHUNK)])
        return carry

    lax.fori_loop(0, CPW, chunk_body, 0)


def kernel(edge_coordinate, edge_index, inverse_sigma1, inverse_sigma2):
    src = edge_index[0].astype(jnp.int32).reshape(E // GROW, GROW)
    dst = edge_index[1].astype(jnp.int32).reshape(E // GROW, GROW)
    sig1 = jnp.broadcast_to(inverse_sigma1.astype(jnp.float32), (16,))
    sig2 = jnp.broadcast_to(inverse_sigma2.astype(jnp.float32), (16,))

    mesh = plsc.VectorSubcoreMesh(core_axis_name="c", subcore_axis_name="s")
    run = pl.kernel(
        _body,
        out_type=jax.ShapeDtypeStruct((E,), jnp.float32),
        mesh=mesh,
        scratch_types=[
            pltpu.VMEM((GPC, GROW), jnp.int32),
            pltpu.VMEM((GPC, GROW), jnp.int32),
            pltpu.VMEM((CHUNK, 6), jnp.float32),
            pltpu.VMEM((CHUNK, 6), jnp.float32),
            pltpu.VMEM((CHUNK,), jnp.float32),
            pltpu.VMEM((16,), jnp.float32),
            pltpu.VMEM((16,), jnp.float32),
            pltpu.SemaphoreType.DMA,
        ],
    )
    return run(edge_coordinate, src, dst, sig1, sig2)


# speedup vs baseline: 45.6600x; 45.6600x over previous
import jax
import jax.numpy as jnp
from jax import lax
from jax.experimental import pallas as pl
from jax.experimental.pallas import tpu as pltpu
from jax.experimental.pallas import tpu_sc as plsc

E = 6_400_000
GROW = 128
CHUNK = 2048
GPC = CHUNK // GROW
NCHUNKS = E // CHUNK
NW = 32
CPW = NCHUNKS // NW
EXTRA = NCHUNKS % NW
VI = CHUNK // 16


def _body(tab_hbm, src_hbm, dst_hbm, sig1_hbm, sig2_hbm, out_hbm,
          idx1_v, idx2_v, rows1_v, rows2_v, out_v, sig1_v, sig2_v, sem):
    nc = 2
    wid = lax.axis_index("s") * nc + lax.axis_index("c")

    pltpu.sync_copy(sig1_hbm, sig1_v)
    pltpu.sync_copy(sig2_hbm, sig2_v)
    s1 = sig1_v[...]
    s2 = sig2_v[...]
    c1 = s1 * s1
    c2 = s2 * s2

    first = wid * CPW + jnp.minimum(wid, EXTRA)
    nchunks = CPW + jnp.where(wid < EXTRA, 1, 0)

    def chunk_body(c, carry):
        cid = first + c
        pltpu.sync_copy(src_hbm.at[cid], idx1_v)
        pltpu.sync_copy(dst_hbm.at[cid], idx2_v)
        copies = []
        for g in range(GPC):
            copies.append(pltpu.async_copy(
                tab_hbm.at[idx1_v.at[g]], rows1_v.at[g], sem))
            copies.append(pltpu.async_copy(
                tab_hbm.at[idx2_v.at[g]], rows2_v.at[g], sem))
        for d in copies:
            d.wait()

        def vec_body(i, carry2):
            rows = i * 16 + lax.iota(jnp.int32, 16)
            gv = jax.lax.shift_right_logical(rows, 7)
            rv = jax.lax.bitwise_and(rows, 127)

            def col(rref, j):
                return plsc.load_gather(
                    rref, [gv, rv, jnp.full((16,), j, jnp.int32)])

            dx = col(rows1_v, 0) - col(rows2_v, 0)
            dy = col(rows1_v, 1) - col(rows2_v, 1)
            dz = col(rows1_v, 2) - col(rows2_v, 2)
            D = dx * dx + dy * dy + dz * dz
            dot = (col(rows1_v, 3) * col(rows2_v, 3)
                   + col(rows1_v, 4) * col(rows2_v, 4)
                   + col(rows1_v, 5) * col(rows2_v, 5))
            t = 1.0 - dot
            T = t * t
            out_v[pl.ds(i * 16, 16)] = jnp.exp(-(D * c1 + T * c2))
            return carry2

        lax.fori_loop(0, VI, vec_body, 0)
        pltpu.sync_copy(out_v, out_hbm.at[pl.ds(cid * CHUNK, CHUNK)])
        return carry

    lax.fori_loop(0, nchunks, chunk_body, 0)


def kernel(edge_coordinate, edge_index, inverse_sigma1, inverse_sigma2):
    n = edge_coordinate.shape[0]
    tab = jnp.concatenate(
        [edge_coordinate.astype(jnp.float32),
         jnp.zeros((n, 2), jnp.float32)], axis=1)
    src = edge_index[0].astype(jnp.int32).reshape(NCHUNKS, GPC, GROW)
    dst = edge_index[1].astype(jnp.int32).reshape(NCHUNKS, GPC, GROW)
    sig1 = jnp.broadcast_to(inverse_sigma1.astype(jnp.float32), (16,))
    sig2 = jnp.broadcast_to(inverse_sigma2.astype(jnp.float32), (16,))

    mesh = plsc.VectorSubcoreMesh(
        core_axis_name="c", subcore_axis_name="s", num_cores=2, num_subcores=16)
    run = pl.kernel(
        _body,
        out_type=jax.ShapeDtypeStruct((E,), jnp.float32),
        mesh=mesh,
        compiler_params=pltpu.CompilerParams(
            needs_layout_passes=False, use_tc_tiling_on_sc=False),
        scratch_types=[
            pltpu.VMEM((GPC, GROW), jnp.int32),
            pltpu.VMEM((GPC, GROW), jnp.int32),
            pltpu.VMEM((GPC, GROW, 8), jnp.float32),
            pltpu.VMEM((GPC, GROW, 8), jnp.float32),
            pltpu.VMEM((CHUNK,), jnp.float32),
            pltpu.VMEM((16,), jnp.float32),
            pltpu.VMEM((16,), jnp.float32),
            pltpu.SemaphoreType.DMA,
        ],
    )
    return run(tab, src, dst, sig1, sig2)


# speedup vs baseline: 46.1332x; 1.0104x over previous
import jax
import jax.numpy as jnp
from jax import lax
from jax.experimental import pallas as pl
from jax.experimental.pallas import tpu as pltpu
from jax.experimental.pallas import tpu_sc as plsc

E = 6_400_000
CHUNK = 2048
NCHUNKS = E // CHUNK
NW = 32
CPW = NCHUNKS // NW
EXTRA = NCHUNKS % NW
VI = CHUNK // 16


def _body(tab_hbm, src_hbm, dst_hbm, sig1_hbm, sig2_hbm, out_hbm,
          idx1_v, idx2_v, rows1_v, rows2_v, out_v, sig1_v, sig2_v, sem):
    nc = 2
    wid = lax.axis_index("s") * nc + lax.axis_index("c")

    pltpu.sync_copy(sig1_hbm, sig1_v)
    pltpu.sync_copy(sig2_hbm, sig2_v)
    s1 = sig1_v[...]
    s2 = sig2_v[...]
    c1 = s1 * s1
    c2 = s2 * s2

    first = wid * CPW + jnp.minimum(wid, EXTRA)
    nchunks = CPW + jnp.where(wid < EXTRA, 1, 0)

    def chunk_body(c, carry):
        e0 = (first + c) * CHUNK
        pltpu.sync_copy(src_hbm.at[pl.ds(e0, CHUNK)], idx1_v)
        pltpu.sync_copy(dst_hbm.at[pl.ds(e0, CHUNK)], idx2_v)
        d1 = pltpu.async_copy(tab_hbm.at[idx1_v], rows1_v, sem)
        d2 = pltpu.async_copy(tab_hbm.at[idx2_v], rows2_v, sem)
        d1.wait()
        d2.wait()

        def vec_body(i, carry2):
            rows = i * 16 + lax.iota(jnp.int32, 16)

            def col(rref, j):
                return plsc.load_gather(
                    rref, [rows, jnp.full((16,), j, jnp.int32)])

            dx = col(rows1_v, 0) - col(rows2_v, 0)
            dy = col(rows1_v, 1) - col(rows2_v, 1)
            dz = col(rows1_v, 2) - col(rows2_v, 2)
            D = dx * dx + dy * dy + dz * dz
            dot = (col(rows1_v, 3) * col(rows2_v, 3)
                   + col(rows1_v, 4) * col(rows2_v, 4)
                   + col(rows1_v, 5) * col(rows2_v, 5))
            t = 1.0 - dot
            T = t * t
            out_v[pl.ds(i * 16, 16)] = jnp.exp(-(D * c1 + T * c2))
            return carry2

        lax.fori_loop(0, VI, vec_body, 0)
        pltpu.sync_copy(out_v, out_hbm.at[pl.ds(e0, CHUNK)])
        return carry

    lax.fori_loop(0, nchunks, chunk_body, 0)


def kernel(edge_coordinate, edge_index, inverse_sigma1, inverse_sigma2):
    n = edge_coordinate.shape[0]
    tab = jnp.concatenate(
        [edge_coordinate.astype(jnp.float32),
         jnp.zeros((n, 2), jnp.float32)], axis=1)
    src = edge_index[0].astype(jnp.int32)
    dst = edge_index[1].astype(jnp.int32)
    sig1 = jnp.broadcast_to(inverse_sigma1.astype(jnp.float32), (16,))
    sig2 = jnp.broadcast_to(inverse_sigma2.astype(jnp.float32), (16,))

    mesh = plsc.VectorSubcoreMesh(
        core_axis_name="c", subcore_axis_name="s", num_cores=2, num_subcores=16)
    run = pl.kernel(
        _body,
        out_type=jax.ShapeDtypeStruct((E,), jnp.float32),
        mesh=mesh,
        compiler_params=pltpu.CompilerParams(
            needs_layout_passes=False, use_tc_tiling_on_sc=False),
        scratch_types=[
            pltpu.VMEM((CHUNK,), jnp.int32),
            pltpu.VMEM((CHUNK,), jnp.int32),
            pltpu.VMEM((CHUNK, 8), jnp.float32),
            pltpu.VMEM((CHUNK, 8), jnp.float32),
            pltpu.VMEM((CHUNK,), jnp.float32),
            pltpu.VMEM((16,), jnp.float32),
            pltpu.VMEM((16,), jnp.float32),
            pltpu.SemaphoreType.DMA,
        ],
    )
    return run(tab, src, dst, sig1, sig2)


# speedup vs baseline: 76.8907x; 1.6667x over previous
import jax
import jax.numpy as jnp
from jax import lax
from jax.experimental import pallas as pl
from jax.experimental.pallas import tpu as pltpu
from jax.experimental.pallas import tpu_sc as plsc

E = 6_400_000
CHUNK = 2048
NCHUNKS = E // CHUNK
NW = 32
CPW = NCHUNKS // NW
EXTRA = NCHUNKS % NW
CMAX = CPW + 1
NPAIR = CMAX // 2
VI = CHUNK // 16


def _body(tab_hbm, src_hbm, dst_hbm, sig1_hbm, sig2_hbm, out_hbm,
          idx1_v0, idx2_v0, idx1_v1, idx2_v1,
          rows1_v0, rows2_v0, rows1_v1, rows2_v1,
          out_v, sig1_v, sig2_v, gsem0, gsem1):
    nc = 2
    wid = lax.axis_index("s") * nc + lax.axis_index("c")

    pltpu.sync_copy(sig1_hbm, sig1_v)
    pltpu.sync_copy(sig2_hbm, sig2_v)
    s1 = sig1_v[...]
    s2 = sig2_v[...]
    c1 = s1 * s1
    c2 = s2 * s2

    first = wid * CPW + jnp.minimum(wid, EXTRA)
    nchunks = CPW + jnp.where(wid < EXTRA, 1, 0)
    last = nchunks - 1

    slots = (
        (idx1_v0, idx2_v0, rows1_v0, rows2_v0, gsem0),
        (idx1_v1, idx2_v1, rows1_v1, rows2_v1, gsem1),
    )

    def issue(c, slot):
        i1, i2, r1, r2, sem = slots[slot]
        e0 = (first + jnp.minimum(c, last)) * CHUNK
        pltpu.sync_copy(src_hbm.at[pl.ds(e0, CHUNK)], i1)
        pltpu.sync_copy(dst_hbm.at[pl.ds(e0, CHUNK)], i2)
        pltpu.async_copy(tab_hbm.at[i1], r1, sem)
        pltpu.async_copy(tab_hbm.at[i2], r2, sem)

    def wait_gathers(slot):
        i1, i2, r1, r2, sem = slots[slot]
        pltpu.make_async_copy(tab_hbm.at[i1], r1, sem).wait()
        pltpu.make_async_copy(tab_hbm.at[i2], r2, sem).wait()

    def compute(c, slot):
        _, _, r1, r2, _ = slots[slot]

        def vec_body(i, carry2):
            rows = i * 16 + lax.iota(jnp.int32, 16)

            def col(rref, j):
                return plsc.load_gather(
                    rref, [rows, jnp.full((16,), j, jnp.int32)])

            dx = col(r1, 0) - col(r2, 0)
            dy = col(r1, 1) - col(r2, 1)
            dz = col(r1, 2) - col(r2, 2)
            D = dx * dx + dy * dy + dz * dz
            dot = (col(r1, 3) * col(r2, 3)
                   + col(r1, 4) * col(r2, 4)
                   + col(r1, 5) * col(r2, 5))
            t = 1.0 - dot
            T = t * t
            out_v[pl.ds(i * 16, 16)] = jnp.exp(-(D * c1 + T * c2))
            return carry2

        lax.fori_loop(0, VI, vec_body, 0)
        e0 = (first + jnp.minimum(c, last)) * CHUNK
        pltpu.sync_copy(out_v, out_hbm.at[pl.ds(e0, CHUNK)])

    issue(0, 0)
    issue(1, 1)

    def pair_body(p, carry):
        for cc in range(2):
            c = 2 * p + cc
            wait_gathers(cc)
            compute(c, cc)
            issue(c + 2, cc)
        return carry

    lax.fori_loop(0, NPAIR, pair_body, 0)
    wait_gathers(0)
    wait_gathers(1)


def kernel(edge_coordinate, edge_index, inverse_sigma1, inverse_sigma2):
    n = edge_coordinate.shape[0]
    tab = jnp.concatenate(
        [edge_coordinate.astype(jnp.float32),
         jnp.zeros((n, 2), jnp.float32)], axis=1)
    src = edge_index[0].astype(jnp.int32)
    dst = edge_index[1].astype(jnp.int32)
    sig1 = jnp.broadcast_to(inverse_sigma1.astype(jnp.float32), (16,))
    sig2 = jnp.broadcast_to(inverse_sigma2.astype(jnp.float32), (16,))

    mesh = plsc.VectorSubcoreMesh(
        core_axis_name="c", subcore_axis_name="s", num_cores=2, num_subcores=16)
    run = pl.kernel(
        _body,
        out_type=jax.ShapeDtypeStruct((E,), jnp.float32),
        mesh=mesh,
        compiler_params=pltpu.CompilerParams(
            needs_layout_passes=False, use_tc_tiling_on_sc=False),
        scratch_types=[
            pltpu.VMEM((CHUNK,), jnp.int32),
            pltpu.VMEM((CHUNK,), jnp.int32),
            pltpu.VMEM((CHUNK,), jnp.int32),
            pltpu.VMEM((CHUNK,), jnp.int32),
            pltpu.VMEM((CHUNK, 8), jnp.float32),
            pltpu.VMEM((CHUNK, 8), jnp.float32),
            pltpu.VMEM((CHUNK, 8), jnp.float32),
            pltpu.VMEM((CHUNK, 8), jnp.float32),
            pltpu.VMEM((CHUNK,), jnp.float32),
            pltpu.VMEM((16,), jnp.float32),
            pltpu.VMEM((16,), jnp.float32),
            pltpu.SemaphoreType.DMA,
            pltpu.SemaphoreType.DMA,
        ],
    )
    return run(tab, src, dst, sig1, sig2)


# speedup vs baseline: 80.6474x; 1.0489x over previous
import jax
import jax.numpy as jnp
from jax import lax
from jax.experimental import pallas as pl
from jax.experimental.pallas import tpu as pltpu
from jax.experimental.pallas import tpu_sc as plsc

E = 6_400_000
CHUNK = 2048
NCHUNKS = E // CHUNK
NW = 32
CPW = NCHUNKS // NW
EXTRA = NCHUNKS % NW
CMAX = CPW + 1
NPAIR = CMAX // 2
VI = CHUNK // 16


def _body(tab_hbm, src_hbm, dst_hbm, sig1_hbm, sig2_hbm, out_hbm,
          idx1_v0, idx2_v0, idx1_v1, idx2_v1,
          rows1_v0, rows2_v0, rows1_v1, rows2_v1,
          out_v, sig1_v, sig2_v, tabs_sh, gsem0, gsem1):
    nc = 2
    sid = lax.axis_index("s")
    wid = sid * nc + lax.axis_index("c")

    @pl.when(sid == 0)
    def _():
        pltpu.sync_copy(tab_hbm, tabs_sh)

    plsc.subcore_barrier()

    pltpu.sync_copy(sig1_hbm, sig1_v)
    pltpu.sync_copy(sig2_hbm, sig2_v)
    s1 = sig1_v[...]
    s2 = sig2_v[...]
    c1 = s1 * s1
    c2 = s2 * s2

    first = wid * CPW + jnp.minimum(wid, EXTRA)
    nchunks = CPW + jnp.where(wid < EXTRA, 1, 0)
    last = nchunks - 1

    slots = (
        (idx1_v0, idx2_v0, rows1_v0, rows2_v0, gsem0),
        (idx1_v1, idx2_v1, rows1_v1, rows2_v1, gsem1),
    )

    def issue(c, slot):
        i1, i2, r1, r2, sem = slots[slot]
        e0 = (first + jnp.minimum(c, last)) * CHUNK
        pltpu.sync_copy(src_hbm.at[pl.ds(e0, CHUNK)], i1)
        pltpu.sync_copy(dst_hbm.at[pl.ds(e0, CHUNK)], i2)
        pltpu.async_copy(tabs_sh.at[i1], r1, sem)
        pltpu.async_copy(tabs_sh.at[i2], r2, sem)

    def wait_gathers(slot):
        i1, i2, r1, r2, sem = slots[slot]
        pltpu.make_async_copy(tabs_sh.at[i1], r1, sem).wait()
        pltpu.make_async_copy(tabs_sh.at[i2], r2, sem).wait()

    def compute(c, slot):
        _, _, r1, r2, _ = slots[slot]

        def vec_body(i, carry2):
            rows = i * 16 + lax.iota(jnp.int32, 16)

            def col(rref, j):
                return plsc.load_gather(
                    rref, [rows, jnp.full((16,), j, jnp.int32)])

            dx = col(r1, 0) - col(r2, 0)
            dy = col(r1, 1) - col(r2, 1)
            dz = col(r1, 2) - col(r2, 2)
            D = dx * dx + dy * dy + dz * dz
            dot = (col(r1, 3) * col(r2, 3)
                   + col(r1, 4) * col(r2, 4)
                   + col(r1, 5) * col(r2, 5))
            t = 1.0 - dot
            T = t * t
            out_v[pl.ds(i * 16, 16)] = jnp.exp(-(D * c1 + T * c2))
            return carry2

        lax.fori_loop(0, VI, vec_body, 0)
        e0 = (first + jnp.minimum(c, last)) * CHUNK
        pltpu.sync_copy(out_v, out_hbm.at[pl.ds(e0, CHUNK)])

    issue(0, 0)
    issue(1, 1)

    def pair_body(p, carry):
        for cc in range(2):
            c = 2 * p + cc
            wait_gathers(cc)
            compute(c, cc)
            issue(c + 2, cc)
        return carry

    lax.fori_loop(0, NPAIR, pair_body, 0)
    wait_gathers(0)
    wait_gathers(1)


def kernel(edge_coordinate, edge_index, inverse_sigma1, inverse_sigma2):
    n = edge_coordinate.shape[0]
    tab = jnp.concatenate(
        [edge_coordinate.astype(jnp.float32),
         jnp.zeros((n, 2), jnp.float32)], axis=1)
    src = edge_index[0].astype(jnp.int32)
    dst = edge_index[1].astype(jnp.int32)
    sig1 = jnp.broadcast_to(inverse_sigma1.astype(jnp.float32), (16,))
    sig2 = jnp.broadcast_to(inverse_sigma2.astype(jnp.float32), (16,))

    mesh = plsc.VectorSubcoreMesh(
        core_axis_name="c", subcore_axis_name="s", num_cores=2, num_subcores=16)
    run = pl.kernel(
        _body,
        out_type=jax.ShapeDtypeStruct((E,), jnp.float32),
        mesh=mesh,
        compiler_params=pltpu.CompilerParams(
            needs_layout_passes=False, use_tc_tiling_on_sc=False),
        scratch_types=[
            pltpu.VMEM((CHUNK,), jnp.int32),
            pltpu.VMEM((CHUNK,), jnp.int32),
            pltpu.VMEM((CHUNK,), jnp.int32),
            pltpu.VMEM((CHUNK,), jnp.int32),
            pltpu.VMEM((CHUNK, 8), jnp.float32),
            pltpu.VMEM((CHUNK, 8), jnp.float32),
            pltpu.VMEM((CHUNK, 8), jnp.float32),
            pltpu.VMEM((CHUNK, 8), jnp.float32),
            pltpu.VMEM((CHUNK,), jnp.float32),
            pltpu.VMEM((16,), jnp.float32),
            pltpu.VMEM((16,), jnp.float32),
            pltpu.VMEM_SHARED((100000, 8), jnp.float32),
            pltpu.SemaphoreType.DMA,
            pltpu.SemaphoreType.DMA,
        ],
    )
    return run(tab, src, dst, sig1, sig2)


# speedup vs baseline: 101.0928x; 1.2535x over previous
import jax
import jax.numpy as jnp
from jax import lax
from jax.experimental import pallas as pl
from jax.experimental.pallas import tpu as pltpu
from jax.experimental.pallas import tpu_sc as plsc

E = 6_400_000
CHUNK = 1024
NCHUNKS = E // CHUNK
NW = 32
CPW = NCHUNKS // NW
EXTRA = NCHUNKS % NW
CMAX = CPW + 1
NQUAD = CMAX // 4
VI = CHUNK // 16


def _body(tab_hbm, src_hbm, dst_hbm, sig1_hbm, sig2_hbm, out_hbm,
          i1b0, i1b1, i1b2, i1b3, i2b0, i2b1, i2b2, i2b3,
          r1b0, r1b1, r1b2, r1b3, r2b0, r2b1, r2b2, r2b3,
          ob0, ob1, ob2, ob3, sig1_v, sig2_v, tabs_sh,
          is0, is1, is2, is3, gs0, gs1, gs2, gs3, os0, os1, os2, os3):
    nc = 2
    sid = lax.axis_index("s")
    wid = sid * nc + lax.axis_index("c")

    @pl.when(sid == 0)
    def _():
        pltpu.sync_copy(tab_hbm, tabs_sh)

    plsc.subcore_barrier()

    pltpu.sync_copy(sig1_hbm, sig1_v)
    pltpu.sync_copy(sig2_hbm, sig2_v)
    s1 = sig1_v[...]
    s2 = sig2_v[...]
    c1 = s1 * s1
    c2 = s2 * s2

    first = wid * CPW + jnp.minimum(wid, EXTRA)
    nchunks = CPW + jnp.where(wid < EXTRA, 1, 0)
    last = nchunks - 1

    idx1 = (i1b0, i1b1, i1b2, i1b3)
    idx2 = (i2b0, i2b1, i2b2, i2b3)
    rows1 = (r1b0, r1b1, r1b2, r1b3)
    rows2 = (r2b0, r2b1, r2b2, r2b3)
    outs = (ob0, ob1, ob2, ob3)
    isems = (is0, is1, is2, is3)
    gsems = (gs0, gs1, gs2, gs3)
    osems = (os0, os1, os2, os3)

    def e0_of(c):
        return (first + jnp.minimum(c, last)) * CHUNK

    def start_idx(c, s):
        e0 = e0_of(c)
        pltpu.async_copy(src_hbm.at[pl.ds(e0, CHUNK)], idx1[s], isems[s])
        pltpu.async_copy(dst_hbm.at[pl.ds(e0, CHUNK)], idx2[s], isems[s])

    def wait_idx(s):
        pltpu.make_async_copy(src_hbm.at[pl.ds(0, CHUNK)], idx1[s],
                              isems[s]).wait()
        pltpu.make_async_copy(dst_hbm.at[pl.ds(0, CHUNK)], idx2[s],
                              isems[s]).wait()

    def fire_gathers(s):
        pltpu.async_copy(tabs_sh.at[idx1[s]], rows1[s], gsems[s])
        pltpu.async_copy(tabs_sh.at[idx2[s]], rows2[s], gsems[s])

    def wait_gathers(s):
        pltpu.make_async_copy(tabs_sh.at[idx1[s]], rows1[s], gsems[s]).wait()
        pltpu.make_async_copy(tabs_sh.at[idx2[s]], rows2[s], gsems[s]).wait()

    def start_out(c, s):
        pltpu.async_copy(outs[s], out_hbm.at[pl.ds(e0_of(c), CHUNK)], osems[s])

    def wait_out(s):
        pltpu.make_async_copy(outs[s], out_hbm.at[pl.ds(0, CHUNK)],
                              osems[s]).wait()

    def compute(s):
        r1, r2, ov = rows1[s], rows2[s], outs[s]

        def vec_body(i, carry2):
            rows = i * 16 + lax.iota(jnp.int32, 16)

            def col(rref, j):
                return plsc.load_gather(
                    rref, [rows, jnp.full((16,), j, jnp.int32)])

            dx = col(r1, 0) - col(r2, 0)
            dy = col(r1, 1) - col(r2, 1)
            dz = col(r1, 2) - col(r2, 2)
            D = dx * dx + dy * dy + dz * dz
            dot = (col(r1, 3) * col(r2, 3)
                   + col(r1, 4) * col(r2, 4)
                   + col(r1, 5) * col(r2, 5))
            t = 1.0 - dot
            T = t * t
            ov[pl.ds(i * 16, 16)] = jnp.exp(-(D * c1 + T * c2))
            return carry2

        lax.fori_loop(0, VI, vec_body, 0)

    for s in range(4):
        start_idx(s, s)
    wait_idx(0)
    fire_gathers(0)
    wait_idx(1)
    fire_gathers(1)

    def quad_body(q, carry):
        for s in range(4):
            c = 4 * q + s
            wait_gathers(s)

            @pl.when(q > 0)
            def _():
                wait_out(s)

            compute(s)
            start_out(c, s)
            start_idx(c + 4, s)
            s2p = (s + 2) % 4
            wait_idx(s2p)
            fire_gathers(s2p)
        return carry

    lax.fori_loop(0, NQUAD, quad_body, 0)

    wait_gathers(0)
    wait_gathers(1)
    wait_idx(2)
    wait_idx(3)
    for s in range(4):
        wait_out(s)


def kernel(edge_coordinate, edge_index, inverse_sigma1, inverse_sigma2):
    n = edge_coordinate.shape[0]
    tab = jnp.concatenate(
        [edge_coordinate.astype(jnp.float32),
         jnp.zeros((n, 2), jnp.float32)], axis=1)
    src = edge_index[0].astype(jnp.int32)
    dst = edge_index[1].astype(jnp.int32)
    sig1 = jnp.broadcast_to(inverse_sigma1.astype(jnp.float32), (16,))
    sig2 = jnp.broadcast_to(inverse_sigma2.astype(jnp.float32), (16,))

    mesh = plsc.VectorSubcoreMesh(
        core_axis_name="c", subcore_axis_name="s", num_cores=2, num_subcores=16)
    run = pl.kernel(
        _body,
        out_type=jax.ShapeDtypeStruct((E,), jnp.float32),
        mesh=mesh,
        compiler_params=pltpu.CompilerParams(
            needs_layout_passes=False, use_tc_tiling_on_sc=False),
        scratch_types=(
            [pltpu.VMEM((CHUNK,), jnp.int32)] * 8
            + [pltpu.VMEM((CHUNK, 8), jnp.float32)] * 8
            + [pltpu.VMEM((CHUNK,), jnp.float32)] * 4
            + [pltpu.VMEM((16,), jnp.float32)] * 2
            + [pltpu.VMEM_SHARED((100000, 8), jnp.float32)]
            + [pltpu.SemaphoreType.DMA] * 12
        ),
    )
    return run(tab, src, dst, sig1, sig2)
